# Initial kernel scaffold; baseline (speedup 1.0000x reference)
import jax
import jax.numpy as jnp
from jax.experimental import pallas as pl


def kernel(idx, targets, table):
    raise NotImplementedError("write your pallas kernel here")


# speedup vs baseline: 1.6908x; 1.6908x over previous
import functools

import jax
import jax.numpy as jnp
from jax import lax
from jax.experimental import pallas as pl
from jax.experimental.pallas import tpu as pltpu
from jax.experimental.pallas import tpu_sc as plsc

_V = 1000
_VP = 1024
_N = 51200
_NC = 2
_NS = 16
_NW = _NC * _NS
_ROWS_W = _N // _NW
_C = 16
_NCHUNK = _ROWS_W // _C
_NBUF = 4
_D = 2
_BLK = 512


def _lse_body(t_ref, o_ref):
    x = t_ref[...]
    m = jnp.max(x, axis=1, keepdims=True)
    s = jnp.sum(jnp.exp(x - m), axis=1, keepdims=True)
    o_ref[...] = m + jnp.log(s)


def _finalize_body(p_ref, o_ref):
    o_ref[...] = (jnp.sum(p_ref[...]) * (1.0 / _N)).reshape(1, 1)


def _compact_body(i_ref, o_ref):
    o_ref[...] = i_ref[:, : _V]


def _make_sc_loss():
    mesh = plsc.VectorSubcoreMesh(core_axis_name="c", subcore_axis_name="s")

    @functools.partial(
        pl.kernel,
        mesh=mesh,
        compiler_params=pltpu.CompilerParams(use_tc_tiling_on_sc=False),
        out_type=jax.ShapeDtypeStruct((_NW, 16), jnp.float32),
        scratch_types=[
            pltpu.VMEM((_ROWS_W,), jnp.int32),
            pltpu.VMEM((_ROWS_W,), jnp.int32),
            pltpu.VMEM((_ROWS_W,), jnp.float32),
            pltpu.VMEM((_ROWS_W,), jnp.float32),
            pltpu.VMEM((16,), jnp.float32),
            pltpu.SemaphoreType.DMA,
        ],
    )
    def sc_loss(idx_hbm, tgt_hbm, lse_hbm, tflat_hbm, part_hbm,
                idx_v, fidx_v, lsei_v, tel_v, acc_v, esem):
        wid = lax.axis_index("s") * _NC + lax.axis_index("c")
        base = wid * _ROWS_W
        pltpu.sync_copy(idx_hbm.at[pl.ds(base, _ROWS_W)], idx_v)
        pltpu.sync_copy(tgt_hbm.at[pl.ds(base, _ROWS_W)], fidx_v)

        def fidx_body(i, carry):
            p = i * 16
            fidx_v[pl.ds(p, 16)] = (fidx_v[pl.ds(p, 16)]
                                    + idx_v[pl.ds(p, 16)] * _V)
            return carry

        lax.fori_loop(0, _ROWS_W // 16, fidx_body, 0)
        pltpu.async_copy(lse_hbm.at[idx_v], lsei_v, esem).wait()
        pltpu.async_copy(tflat_hbm.at[fidx_v], tel_v, esem).wait()
        acc_v[...] = jnp.zeros((16,), jnp.float32)

        def acc_body(i, carry):
            p = i * 16
            acc_v[...] = acc_v[...] + (lsei_v[pl.ds(p, 16)]
                                       - tel_v[pl.ds(p, 16)])
            return carry

        lax.fori_loop(0, _ROWS_W // 16, acc_body, 0)
        pltpu.sync_copy(acc_v, part_hbm.at[wid])

    return sc_loss


def _make_sc_gather():
    mesh = plsc.VectorSubcoreMesh(core_axis_name="c", subcore_axis_name="s")

    @functools.partial(
        pl.kernel,
        mesh=mesh,
        out_type=jax.ShapeDtypeStruct((_N, _VP), jnp.float32),
        scratch_types=(
            [pltpu.VMEM((_ROWS_W,), jnp.int32)]
            + [pltpu.VMEM((_C, _VP), jnp.float32) for _ in range(_NBUF)]
            + [pltpu.SemaphoreType.DMA for _ in range(2 * _NBUF)]
        ),
    )
    def sc_gather(idx_hbm, tpad_hbm, out_hbm, idx_v, *rest):
        bufs = rest[:_NBUF]
        gsems = rest[_NBUF:2 * _NBUF]
        wsems = rest[2 * _NBUF:3 * _NBUF]

        wid = lax.axis_index("s") * _NC + lax.axis_index("c")
        base = wid * _ROWS_W
        pltpu.sync_copy(idx_hbm.at[pl.ds(base, _ROWS_W)], idx_v)

        def gather_desc(g, b):
            return pltpu.make_async_copy(
                tpad_hbm.at[idx_v.at[pl.ds(g * _C, _C)]], bufs[b], gsems[b])

        def write_desc(g, b):
            return pltpu.make_async_copy(
                bufs[b], out_hbm.at[pl.ds(base + g * _C, _C)], wsems[b])

        def do_chunk(g, b):
            gather_desc(g, b).wait()
            write_desc(g, b).start()
            bf = (b + _D) % _NBUF
            f = g + _D

            @pl.when(f < _NCHUNK)
            def _():
                @pl.when(f >= _NBUF)
                def _():
                    write_desc(f - _NBUF, bf).wait()
                gather_desc(f, bf).start()

        for p in range(_D):
            gather_desc(p, p).start()

        def body(i, carry):
            g0 = i * _NBUF
            for b in range(_NBUF):
                do_chunk(g0 + b, b)
            return carry

        lax.fori_loop(0, _NCHUNK // _NBUF, body, 0)
        for t in range(_NCHUNK - _NBUF + _D, _NCHUNK):
            write_desc(t, t % _NBUF).wait()

    return sc_gather


_sc_loss = _make_sc_loss()
_sc_gather = _make_sc_gather()


def kernel(idx, targets, table):
    idxf = idx.reshape(-1).astype(jnp.int32)
    tgtf = targets.reshape(-1).astype(jnp.int32)
    lse = pl.pallas_call(
        _lse_body,
        out_shape=jax.ShapeDtypeStruct((_V, 1), jnp.float32),
    )(table).reshape(_V)
    tpad = jnp.pad(table, ((0, 0), (0, _VP - _V)))
    tflat = jnp.pad(table.reshape(-1), (0, 8))
    part = _sc_loss(idxf, tgtf, lse, tflat)
    out_pad = _sc_gather(idxf, tpad)
    logits2d = pl.pallas_call(
        _compact_body,
        grid=(_N // _BLK,),
        in_specs=[pl.BlockSpec((_BLK, _VP), lambda i: (i, 0))],
        out_specs=pl.BlockSpec((_BLK, _V), lambda i: (i, 0)),
        out_shape=jax.ShapeDtypeStruct((_N, _V), jnp.float32),
    )(out_pad)
    loss = pl.pallas_call(
        _finalize_body,
        out_shape=jax.ShapeDtypeStruct((1, 1), jnp.float32),
    )(part)
    return (logits2d, loss.reshape(()))


# speedup vs baseline: 2.4311x; 1.4378x over previous
import functools

import jax
import jax.numpy as jnp
from jax import lax
from jax.experimental import pallas as pl
from jax.experimental.pallas import tpu as pltpu
from jax.experimental.pallas import tpu_sc as plsc

_V = 1000
_VP = 1024
_N = 51200
_NC = 2
_NS = 16
_NW = _NC * _NS
_ROWS_W = _N // _NW
_C = 16
_NCHUNK = _ROWS_W // _C
_NBUF = 4
_D = 2
_BLK = 512


def _lse_body(t_ref, o_ref):
    x = t_ref[...]
    m = jnp.max(x, axis=1, keepdims=True)
    s = jnp.sum(jnp.exp(x - m), axis=1, keepdims=True)
    o_ref[...] = m + jnp.log(s)


def _finalize_body(p_ref, o_ref):
    o_ref[...] = (jnp.sum(p_ref[...]) * (1.0 / _N)).reshape(1, 1)


def _compact_body(i_ref, o_ref):
    o_ref[...] = i_ref[:, : _V].T


def _make_sc_gather():
    mesh = plsc.VectorSubcoreMesh(core_axis_name="c", subcore_axis_name="s")

    @functools.partial(
        pl.kernel,
        mesh=mesh,
        out_type=[jax.ShapeDtypeStruct((_N, _VP), jnp.float32),
                  jax.ShapeDtypeStruct((_NW, 16), jnp.float32)],
        scratch_types=(
            [pltpu.VMEM((_ROWS_W,), jnp.int32),
             pltpu.VMEM((_ROWS_W,), jnp.int32),
             pltpu.VMEM((_ROWS_W,), jnp.float32),
             pltpu.VMEM((_ROWS_W,), jnp.float32),
             pltpu.VMEM((16,), jnp.float32)]
            + [pltpu.VMEM((_C, _VP), jnp.float32) for _ in range(_NBUF)]
            + [pltpu.SemaphoreType.DMA, pltpu.SemaphoreType.DMA]
            + [pltpu.SemaphoreType.DMA for _ in range(2 * _NBUF)]
        ),
    )
    def sc_gather(idx_hbm, tgt_hbm, lse_hbm, tflat_hbm, tpad_hbm,
                  out_hbm, part_hbm,
                  idx_v, fidx_v, lsei_v, tel_v, acc_v, *rest):
        bufs = rest[:_NBUF]
        esem1 = rest[_NBUF]
        esem2 = rest[_NBUF + 1]
        gsems = rest[_NBUF + 2:2 * _NBUF + 2]
        wsems = rest[2 * _NBUF + 2:3 * _NBUF + 2]

        wid = lax.axis_index("s") * _NC + lax.axis_index("c")
        base = wid * _ROWS_W
        pltpu.sync_copy(idx_hbm.at[pl.ds(base, _ROWS_W)], idx_v)
        pltpu.sync_copy(tgt_hbm.at[pl.ds(base, _ROWS_W)], fidx_v)

        def fidx_body(i, carry):
            p = i * 16
            fidx_v[pl.ds(p, 16)] = (fidx_v[pl.ds(p, 16)]
                                    + idx_v[pl.ds(p, 16)] * _V)
            return carry

        lax.fori_loop(0, _ROWS_W // 16, fidx_body, 0)
        e1 = pltpu.make_async_copy(lse_hbm.at[idx_v], lsei_v, esem1)
        e1.start()
        e2 = pltpu.make_async_copy(tflat_hbm.at[fidx_v], tel_v, esem2)
        e2.start()

        def gather_desc(g, b):
            return pltpu.make_async_copy(
                tpad_hbm.at[idx_v.at[pl.ds(g * _C, _C)]], bufs[b], gsems[b])

        def write_desc(g, b):
            return pltpu.make_async_copy(
                bufs[b], out_hbm.at[pl.ds(base + g * _C, _C)], wsems[b])

        def do_chunk(g, b):
            gather_desc(g, b).wait()
            write_desc(g, b).start()
            bf = (b + _D) % _NBUF
            f = g + _D

            @pl.when(f < _NCHUNK)
            def _():
                @pl.when(f >= _NBUF)
                def _():
                    write_desc(f - _NBUF, bf).wait()
                gather_desc(f, bf).start()

        for p in range(_D):
            gather_desc(p, p).start()

        def body(i, carry):
            g0 = i * _NBUF
            for b in range(_NBUF):
                do_chunk(g0 + b, b)
            return carry

        lax.fori_loop(0, _NCHUNK // _NBUF, body, 0)
        for t in range(_NCHUNK - _NBUF + _D, _NCHUNK):
            write_desc(t, t % _NBUF).wait()

        e1.wait()
        e2.wait()
        acc_v[...] = jnp.zeros((16,), jnp.float32)

        def acc_body(i, carry):
            p = i * 16
            acc_v[...] = acc_v[...] + (lsei_v[pl.ds(p, 16)]
                                       - tel_v[pl.ds(p, 16)])
            return carry

        lax.fori_loop(0, _ROWS_W // 16, acc_body, 0)
        pltpu.sync_copy(acc_v, part_hbm.at[wid])

    return sc_gather


_sc_gather = _make_sc_gather()


def kernel(idx, targets, table):
    idxf = idx.reshape(-1).astype(jnp.int32)
    tgtf = targets.reshape(-1).astype(jnp.int32)
    lse = pl.pallas_call(
        _lse_body,
        out_shape=jax.ShapeDtypeStruct((_V, 1), jnp.float32),
    )(table).reshape(_V)
    tpad = jnp.pad(table, ((0, 0), (0, _VP - _V)))
    tflat = jnp.pad(table.reshape(-1), (0, 8))
    out_pad, part = _sc_gather(idxf, tgtf, lse, tflat, tpad)
    logits2d_t = pl.pallas_call(
        _compact_body,
        grid=(_N // _BLK,),
        in_specs=[pl.BlockSpec((_BLK, _VP), lambda i: (i, 0))],
        out_specs=pl.BlockSpec((_V, _BLK), lambda i: (0, i)),
        out_shape=jax.ShapeDtypeStruct((_V, _N), jnp.float32),
    )(out_pad)
    logits2d = logits2d_t.T
    loss = pl.pallas_call(
        _finalize_body,
        out_shape=jax.ShapeDtypeStruct((1, 1), jnp.float32),
    )(part)
    return (logits2d, loss.reshape(()))
